# baseline (speedup 1.0000x reference)
import functools

import jax
import jax.numpy as jnp
from jax import lax
from jax.experimental import pallas as pl
from jax.experimental.pallas import tpu as pltpu
from jax.experimental.pallas import tpu_sc as plsc

VOCAB = 1000000
EMBED = 32
SEQ = 50
SEQ_PAD = 64
BATCH = 4096
ENC_UNITS = 256

N_IDX = BATCH * SEQ_PAD
NUM_CORES = 2
NUM_SUBCORES = 16
NUM_WORKERS = NUM_CORES * NUM_SUBCORES

X_ROWS_PER_WORKER = BATCH // NUM_WORKERS
IDX_PER_WORKER = N_IDX // NUM_WORKERS
CHUNK = 1024
NUM_CHUNKS = IDX_PER_WORKER // CHUNK

_MESH = plsc.VectorSubcoreMesh(
    core_axis_name="c", subcore_axis_name="s",
    num_cores=NUM_CORES, num_subcores=NUM_SUBCORES)


def _worker_id():
    return lax.axis_index("s") * NUM_CORES + lax.axis_index("c")


def _spread_body(x_hbm, idx_hbm, xv2, sem):
    wid = _worker_id()
    base = wid * X_ROWS_PER_WORKER
    pltpu.sync_copy(x_hbm.at[pl.ds(base, X_ROWS_PER_WORKER), :], xv2)
    copies = []
    for j in range(X_ROWS_PER_WORKER):
        copies.append(pltpu.async_copy(
            xv2.at[j, :],
            idx_hbm.at[pl.ds((base + j) * SEQ_PAD, SEQ)],
            sem))
    for c in copies:
        c.wait()


_spread = pl.kernel(
    _spread_body,
    out_type=jax.ShapeDtypeStruct((N_IDX,), jnp.int32),
    mesh=_MESH,
    scratch_types=[
        pltpu.VMEM((X_ROWS_PER_WORKER, SEQ), jnp.int32),
        pltpu.SemaphoreType.DMA,
    ],
)


def _gather_body(table_hbm, idx_hbm, out_hbm, idx_v, rows_a, rows_b,
                 sem_g, sem_wa, sem_wb):
    wid = _worker_id()
    base = wid * IDX_PER_WORKER
    rows = (rows_a, rows_b)
    sems = (sem_wa, sem_wb)
    writes = [None, None]
    for i in range(NUM_CHUNKS):
        off = base + i * CHUNK
        pltpu.sync_copy(idx_hbm.at[pl.ds(off, CHUNK)], idx_v)

        def clamp(k, _):
            v = idx_v[pl.ds(k * 16, 16)]
            lane = lax.broadcasted_iota(jnp.int32, (16,), 0)
            pos = (k % 4) * 16 + lane
            slot = off + k * 16 + lane
            v = jnp.minimum(jnp.maximum(v, 0), VOCAB - 1)
            v = jnp.where(pos >= SEQ, slot, v)
            idx_v[pl.ds(k * 16, 16)] = v
            return ()
        lax.fori_loop(0, CHUNK // 16, clamp, ())

        buf = i % 2
        if writes[buf] is not None:
            writes[buf].wait()
        pltpu.async_copy(table_hbm.at[idx_v], rows[buf], sem_g).wait()
        writes[buf] = pltpu.async_copy(
            rows[buf], out_hbm.at[pl.ds(off, CHUNK), :], sems[buf])
    for w in writes:
        if w is not None:
            w.wait()


_gather = pl.kernel(
    _gather_body,
    out_type=jax.ShapeDtypeStruct((N_IDX, EMBED), jnp.float32),
    mesh=_MESH,
    scratch_types=[
        pltpu.VMEM((CHUNK,), jnp.int32),
        pltpu.VMEM((CHUNK, EMBED), jnp.float32),
        pltpu.VMEM((CHUNK, EMBED), jnp.float32),
        pltpu.SemaphoreType.DMA,
        pltpu.SemaphoreType.DMA,
        pltpu.SemaphoreType.DMA,
    ],
    compiler_params=pltpu.CompilerParams(use_tc_tiling_on_sc=False),
)


DEPAD_BLK = 800


def _depad_body(t_ref, o_ref):
    t = t_ref[...].reshape(DEPAD_BLK // 4, 4, EMBED)
    for k in range(4):
        o_ref[:, k * EMBED:(k + 1) * EMBED] = t[:, k, :]


_depad = pl.pallas_call(
    _depad_body,
    grid=(VOCAB // DEPAD_BLK,),
    in_specs=[pl.BlockSpec((DEPAD_BLK, EMBED), lambda i: (i, 0))],
    out_specs=pl.BlockSpec((DEPAD_BLK // 4, 4 * EMBED), lambda i: (i, 0)),
    out_shape=jax.ShapeDtypeStruct((VOCAB // 4, 4 * EMBED), jnp.float32),
)


BB = 512


def _mlp_body(flat_ref, w_ref, b_ref, out_ref):
    acc = jnp.dot(flat_ref[:, :SEQ * EMBED], w_ref[...],
                  preferred_element_type=jnp.float32)
    out_ref[...] = jnp.tanh(acc + b_ref[...])


_mlp = pl.pallas_call(
    _mlp_body,
    grid=(BATCH // BB,),
    in_specs=[
        pl.BlockSpec((BB, SEQ_PAD * EMBED), lambda i: (i, 0)),
        pl.BlockSpec((SEQ * EMBED, ENC_UNITS), lambda i: (0, 0)),
        pl.BlockSpec((1, ENC_UNITS), lambda i: (0, 0)),
    ],
    out_specs=pl.BlockSpec((BB, ENC_UNITS), lambda i: (i, 0)),
    out_shape=jax.ShapeDtypeStruct((BATCH, ENC_UNITS), jnp.float32),
)


def kernel(x, table, W, b):
    idx = _spread(x.astype(jnp.int32))
    table_lin = _depad(table).reshape(VOCAB, EMBED)
    rows = _gather(table_lin, idx)
    flat = rows.reshape(BATCH, SEQ_PAD * EMBED)
    return _mlp(flat, W, b.reshape(1, ENC_UNITS))


# speedup vs baseline: 2.0894x; 2.0894x over previous
import functools

import jax
import jax.numpy as jnp
from jax import lax
from jax.experimental import pallas as pl
from jax.experimental.pallas import tpu as pltpu
from jax.experimental.pallas import tpu_sc as plsc

VOCAB = 1000000
EMBED = 32
SEQ = 50
SEQ_PAD = 64
BATCH = 4096
ENC_UNITS = 256

N_IDX = BATCH * SEQ_PAD
NUM_CORES = 2
NUM_SUBCORES = 16
NUM_WORKERS = NUM_CORES * NUM_SUBCORES

X_ROWS_PER_WORKER = BATCH // NUM_WORKERS
IDX_PER_WORKER = N_IDX // NUM_WORKERS
CHUNK = 1024
NUM_CHUNKS = IDX_PER_WORKER // CHUNK

_MESH = plsc.VectorSubcoreMesh(
    core_axis_name="c", subcore_axis_name="s",
    num_cores=NUM_CORES, num_subcores=NUM_SUBCORES)


def _worker_id():
    return lax.axis_index("s") * NUM_CORES + lax.axis_index("c")


def _spread_body(x_hbm, idx_hbm, xv2, sem):
    wid = _worker_id()
    base = wid * X_ROWS_PER_WORKER
    pltpu.sync_copy(x_hbm.at[pl.ds(base, X_ROWS_PER_WORKER), :], xv2)
    copies = []
    for j in range(X_ROWS_PER_WORKER):
        copies.append(pltpu.async_copy(
            xv2.at[j, :],
            idx_hbm.at[pl.ds((base + j) * SEQ_PAD, SEQ)],
            sem))
    for c in copies:
        c.wait()


_spread = pl.kernel(
    _spread_body,
    out_type=jax.ShapeDtypeStruct((N_IDX,), jnp.int32),
    mesh=_MESH,
    scratch_types=[
        pltpu.VMEM((X_ROWS_PER_WORKER, SEQ), jnp.int32),
        pltpu.SemaphoreType.DMA,
    ],
)


def _gather_body(table_hbm, idx_hbm, out_hbm, idx_v, rows_a, rows_b,
                 sem_g, sem_wa, sem_wb):
    wid = _worker_id()
    base = wid * IDX_PER_WORKER
    rows = (rows_a, rows_b)
    sems = (sem_wa, sem_wb)
    writes = [None, None]
    for i in range(NUM_CHUNKS):
        off = base + i * CHUNK
        pltpu.sync_copy(idx_hbm.at[pl.ds(off, CHUNK)], idx_v)

        def clamp(k, _):
            v = idx_v[pl.ds(k * 16, 16)]
            lane = lax.broadcasted_iota(jnp.int32, (16,), 0)
            pos = (k % 4) * 16 + lane
            slot = off + k * 16 + lane
            v = jnp.minimum(jnp.maximum(v, 0), VOCAB - 1)
            v = jnp.where(pos >= SEQ, slot, v)
            idx_v[pl.ds(k * 16, 16)] = v
            return ()
        lax.fori_loop(0, CHUNK // 16, clamp, ())

        buf = i % 2
        if writes[buf] is not None:
            writes[buf].wait()
        pltpu.async_copy(table_hbm.at[idx_v], rows[buf], sem_g).wait()
        writes[buf] = pltpu.async_copy(
            rows[buf], out_hbm.at[pl.ds(off, CHUNK), :], sems[buf])
    for w in writes:
        if w is not None:
            w.wait()


_gather = pl.kernel(
    _gather_body,
    out_type=jax.ShapeDtypeStruct((N_IDX, EMBED), jnp.float32),
    mesh=_MESH,
    scratch_types=[
        pltpu.VMEM((CHUNK,), jnp.int32),
        pltpu.VMEM((CHUNK, EMBED), jnp.float32),
        pltpu.VMEM((CHUNK, EMBED), jnp.float32),
        pltpu.SemaphoreType.DMA,
        pltpu.SemaphoreType.DMA,
        pltpu.SemaphoreType.DMA,
    ],
    compiler_params=pltpu.CompilerParams(use_tc_tiling_on_sc=False),
)


BB = 512


def _mlp_body(flat_ref, w_ref, b_ref, out_ref):
    acc = jnp.dot(flat_ref[:, :SEQ * EMBED], w_ref[...],
                  preferred_element_type=jnp.float32)
    out_ref[...] = jnp.tanh(acc + b_ref[...])


_mlp = pl.pallas_call(
    _mlp_body,
    grid=(BATCH // BB,),
    in_specs=[
        pl.BlockSpec((BB, SEQ_PAD * EMBED), lambda i: (i, 0)),
        pl.BlockSpec((SEQ * EMBED, ENC_UNITS), lambda i: (0, 0)),
        pl.BlockSpec((1, ENC_UNITS), lambda i: (0, 0)),
    ],
    out_specs=pl.BlockSpec((BB, ENC_UNITS), lambda i: (i, 0)),
    out_shape=jax.ShapeDtypeStruct((BATCH, ENC_UNITS), jnp.float32),
)


def kernel(x, table, W, b):
    idx = _spread(x.astype(jnp.int32))
    rows = _gather(table, idx)
    flat = rows.reshape(BATCH, SEQ_PAD * EMBED)
    return _mlp(flat, W, b.reshape(1, ENC_UNITS))
